# baseline (speedup 1.0000x reference)
import functools

import jax
import jax.numpy as jnp
from jax import lax
from jax.experimental import pallas as pl
from jax.experimental.pallas import tpu as pltpu
from jax.experimental.pallas import tpu_sc as plsc

N = 10000
E = 320000
D = 64
D2 = 128
L = 14

NC = 2
NS = 16
NW = NC * NS
EPT = E // NW
G = 80
NCHK = EPT // G
ZR = 80
NZC = N // ZR

_mesh = plsc.VectorSubcoreMesh(core_axis_name="c", subcore_axis_name="s")


def _edge_body(h_hbm, ea_hbm, src_hbm, dst_hbm, t_hbm, z_hbm,
               pn_hbm, pd_hbm,
               rows0, rows1, rows2, rows3, ea0, ea1, ea2, ea3,
               srcs0, srcs1, srcs2, srcs3, dsts0, dsts1, dsts2, dsts3,
               t_v,
               accn, accd,
               gsem0, gsem1, ssem0, ssem1,
               xsem0, xsem1, xsem2, xsem3, dsem0, dsem1, isem):
    cid = lax.axis_index("c")
    sid = lax.axis_index("s")
    wid = cid * NS + sid
    ebase = wid * EPT
    rbase = wid * NCHK

    rows = (rows0, rows1, rows2, rows3)
    eav = (ea0, ea1, ea2, ea3)
    srcs = (srcs0, srcs1, srcs2, srcs3)
    dsts = (dsts0, dsts1, dsts2, dsts3)
    gsem = (gsem0, gsem1)
    ssem = (ssem0, ssem1)
    xsem = (xsem0, xsem1, xsem2, xsem3)
    dsem = (dsem0, dsem1)

    def _issue_src(c, sl):
        pltpu.async_copy(src_hbm.at[rbase + c], srcs[sl], xsem[sl])

    def _wait_src(c, sl):
        pltpu.make_async_copy(src_hbm.at[rbase + c], srcs[sl], xsem[sl]).wait()

    def _issue_dst(c, sl, sp):
        pltpu.async_copy(dst_hbm.at[rbase + c], dsts[sl], dsem[sp])

    def _wait_dst(c, sl, sp):
        pltpu.make_async_copy(dst_hbm.at[rbase + c], dsts[sl], dsem[sp]).wait()

    def _issue_in(c, q, p):
        pltpu.async_copy(ea_hbm.at[pl.ds(ebase + c * G, G)], eav[q], gsem[p])
        pltpu.async_copy(h_hbm.at[srcs[q]], rows[q], gsem[p])

    def _wait_in(c, q, p):
        pltpu.make_async_copy(ea_hbm.at[pl.ds(ebase + c * G, G)], eav[q], gsem[p]).wait()
        pltpu.make_async_copy(h_hbm.at[srcs[q]], rows[q], gsem[p]).wait()

    def _issue_scatter(c, q, p):
        pltpu.async_copy(rows[q], accn.at[dsts[q]], ssem[p], add=True)
        pltpu.async_copy(eav[q], accd.at[dsts[q]], ssem[p], add=True)

    def _wait_scatter(c, q, p):
        pltpu.make_async_copy(rows[q], accn.at[dsts[q]], ssem[p]).wait()
        pltpu.make_async_copy(eav[q], accd.at[dsts[q]], ssem[p]).wait()

    pltpu.sync_copy(t_hbm, t_v)
    for c in range(4):
        _issue_src(c, c)
    _issue_dst(0, 0, 0)
    _issue_dst(1, 1, 1)

    for k in range((NZC + NS - 1) // NS):
        c = sid + k * NS

        @pl.when(c < NZC)
        def _zero_issue(c=c):
            pltpu.async_copy(z_hbm, accn.at[pl.ds(c * ZR, ZR)], isem)
            pltpu.async_copy(z_hbm, accd.at[pl.ds(c * ZR, ZR)], isem)

    _wait_src(0, 0)
    _issue_in(0, 0, 0)
    _wait_src(1, 1)
    _issue_in(1, 1, 1)

    for k in range((NZC + NS - 1) // NS):
        c = sid + k * NS

        @pl.when(c < NZC)
        def _zero_wait(c=c):
            pltpu.make_async_copy(z_hbm, accn.at[pl.ds(c * ZR, ZR)], isem).wait()
            pltpu.make_async_copy(z_hbm, accd.at[pl.ds(c * ZR, ZR)], isem).wait()

    plsc.subcore_barrier()

    t = t_v[...]

    def _process(c, k):
        q = k
        p = k % 2
        _wait_in(c, q, p)

        def _drain(c=c, k=k, p=p):
            _wait_scatter(c - 2, (k - 2) % 4, p)

        if isinstance(c, int):
            if c >= 2:
                _drain()
        else:
            pl.when(c >= 2)(_drain)

        rv, ev = rows[q], eav[q]

        @plsc.parallel_loop(0, G, 1, unroll=8)
        def _edge(e):
            for j in range(4):
                sl = pl.ds(j * 16, 16)
                m = jnp.maximum(rv[e, sl] + ev[e, sl], 0.0) + 1e-7
                s = jnp.exp(m * t)
                rv[e, sl] = s * m
                ev[e, sl] = s

        _wait_dst(c, k, p)
        _issue_scatter(c, q, p)

        def _pf(c=c, k=k, p=p):
            _wait_src(c + 2, (k + 2) % 4)
            _issue_in(c + 2, (k + 2) % 4, p)
            _issue_dst(c + 2, (k + 2) % 4, p)

        def _pf4(c=c, k=k):
            _issue_src(c + 4, k)

        if isinstance(c, int):
            if c + 2 < NCHK:
                _pf()
            if c + 4 < NCHK:
                _pf4()
        else:
            pl.when(c + 2 < NCHK)(_pf)
            pl.when(c + 4 < NCHK)(_pf4)

    def _step(i, carry):
        base = 4 * i
        for k in range(4):
            _process(base + k, k)
        return carry
    lax.fori_loop(0, NCHK // 4, _step, 0)

    _process(NCHK - 1, (NCHK - 1) % 4)
    _wait_scatter(NCHK - 2, (NCHK - 2) % 4, (NCHK - 2) % 2)
    _wait_scatter(NCHK - 1, (NCHK - 1) % 4, (NCHK - 1) % 2)
    plsc.subcore_barrier()

    for k in range((NZC + NS - 1) // NS):
        c = sid + k * NS

        @pl.when(c < NZC)
        def _read_issue(c=c):
            pltpu.async_copy(accn.at[pl.ds(c * ZR, ZR)],
                             pn_hbm.at[cid, pl.ds(c * ZR, ZR)], isem)
            pltpu.async_copy(accd.at[pl.ds(c * ZR, ZR)],
                             pd_hbm.at[cid, pl.ds(c * ZR, ZR)], isem)

    for k in range((NZC + NS - 1) // NS):
        c = sid + k * NS

        @pl.when(c < NZC)
        def _read_wait(c=c):
            pltpu.make_async_copy(accn.at[pl.ds(c * ZR, ZR)],
                                  pn_hbm.at[cid, pl.ds(c * ZR, ZR)], isem).wait()
            pltpu.make_async_copy(accd.at[pl.ds(c * ZR, ZR)],
                                  pd_hbm.at[cid, pl.ds(c * ZR, ZR)], isem).wait()


_edge_call = functools.partial(
    pl.kernel,
    out_type=[jax.ShapeDtypeStruct((NC, N, D), jnp.float32),
              jax.ShapeDtypeStruct((NC, N, D), jnp.float32)],
    mesh=_mesh,
    compiler_params=pltpu.CompilerParams(use_tc_tiling_on_sc=False),
    scratch_types=(
        [pltpu.VMEM((G, D), jnp.float32) for _ in range(8)]
        + [pltpu.VMEM((G,), jnp.int32) for _ in range(8)]
        + [pltpu.VMEM((16,), jnp.float32)]
        + [pltpu.VMEM_SHARED((N, D), jnp.float32),
           pltpu.VMEM_SHARED((N, D), jnp.float32)]
        + [pltpu.SemaphoreType.DMA for _ in range(11)]
    ),
)(_edge_body)


def _ln(h, g, b, eps=1e-5):
    mu = jnp.mean(h, axis=-1, keepdims=True)
    var = jnp.var(h, axis=-1, keepdims=True)
    return (h - mu) * lax.rsqrt(var + eps) * g + b


NB = 1000


def _encode_body(x_ref, w_ref, b_ref, o_ref):
    o_ref[...] = jnp.dot(x_ref[...], w_ref[...],
                         preferred_element_type=jnp.float32) + b_ref[...]


def _encode(x, w, b):
    rows = x.shape[0]
    return pl.pallas_call(
        _encode_body,
        grid=(rows // NB,),
        in_specs=[
            pl.BlockSpec((NB, D2), lambda i: (i, 0)),
            pl.BlockSpec((D2, D), lambda i: (0, 0)),
            pl.BlockSpec((1, D), lambda i: (0, 0)),
        ],
        out_specs=pl.BlockSpec((NB, D), lambda i: (i, 0)),
        out_shape=jax.ShapeDtypeStruct((rows, D), jnp.float32),
    )(x, w, b)


def _node_body(pn_ref, pd_ref, cin_ref, hprev_ref,
               w1_ref, b1_ref, g1_ref, bt1_ref, w2_ref, b2_ref,
               gn_ref, bn_ref, hout_ref, nin_ref):
    numer = pn_ref[0] + pn_ref[1]
    denom = pd_ref[0] + pd_ref[1]
    out = numer / (denom + 1e-16) + cin_ref[...]
    hm = jnp.dot(out, w1_ref[...], preferred_element_type=jnp.float32) + b1_ref[...]
    hm = jax.nn.relu(_ln(hm, g1_ref[...], bt1_ref[...]))
    r = jnp.dot(hm, w2_ref[...], preferred_element_type=jnp.float32) + b2_ref[...]
    h = hprev_ref[...] + r
    hout_ref[...] = h
    nin_ref[...] = jax.nn.relu(_ln(h, gn_ref[...], bn_ref[...]))


def _node(pn, pd, cin, hprev, w1, b1, g1, bt1, w2, b2, gn, bn):
    vec = lambda: pl.BlockSpec((1, D2), lambda i: (0, 0))
    vec64 = lambda: pl.BlockSpec((1, D), lambda i: (0, 0))
    return pl.pallas_call(
        _node_body,
        grid=(N // NB,),
        in_specs=[
            pl.BlockSpec((NC, NB, D), lambda i: (0, i, 0)),
            pl.BlockSpec((NC, NB, D), lambda i: (0, i, 0)),
            pl.BlockSpec((NB, D), lambda i: (i, 0)),
            pl.BlockSpec((NB, D), lambda i: (i, 0)),
            pl.BlockSpec((D, D2), lambda i: (0, 0)),
            vec(), vec(), vec(),
            pl.BlockSpec((D2, D), lambda i: (0, 0)),
            vec64(), vec64(), vec64(),
        ],
        out_specs=[pl.BlockSpec((NB, D), lambda i: (i, 0)),
                   pl.BlockSpec((NB, D), lambda i: (i, 0))],
        out_shape=[jax.ShapeDtypeStruct((N, D), jnp.float32),
                   jax.ShapeDtypeStruct((N, D), jnp.float32)],
    )(pn, pd, cin, hprev, w1, b1, g1, bt1, w2, b2, gn, bn)


def kernel(x, edge_index, edge_attr, W_ne, b_ne, W_ee, b_ee, t, W1, b1, g1, bt1, W2, b2, ln_g, ln_b):
    src = edge_index[0].reshape(E // G, G)
    dst = edge_index[1].reshape(E // G, G)
    h0 = _encode(x, W_ne, b_ne.reshape(1, D))
    ea = _encode(edge_attr, W_ee, b_ee.reshape(1, D))
    tvecs = jnp.broadcast_to(t[:, None], (L, 16)).astype(jnp.float32)
    zblock = jnp.zeros((ZR, D), jnp.float32)

    cin = h0
    hprev = jnp.zeros((N, D), jnp.float32)
    for i in range(L):
        pn, pd = _edge_call(cin, ea, src, dst, tvecs[i], zblock)
        j = (i + 1) % L
        hprev, cin = _node(pn, pd, cin, hprev,
                           W1[i], b1[i].reshape(1, D2), g1[i].reshape(1, D2),
                           bt1[i].reshape(1, D2), W2[i], b2[i].reshape(1, D),
                           ln_g[j].reshape(1, D), ln_b[j].reshape(1, D))
    return cin


# speedup vs baseline: 1.2084x; 1.2084x over previous
import functools

import jax
import jax.numpy as jnp
from jax import lax
from jax.experimental import pallas as pl
from jax.experimental.pallas import tpu as pltpu
from jax.experimental.pallas import tpu_sc as plsc

N = 10000
E = 320000
D = 64
D2 = 128
L = 14

NC = 2
NS = 16
NW = NC * NS
EPT = E // NW
G = 50
NCHK = EPT // G
ZR = N // NS

_mesh = plsc.VectorSubcoreMesh(core_axis_name="c", subcore_axis_name="s")


def _edge_body(h_hbm, ea_hbm, src_hbm, dst_hbm, t_hbm, z_hbm, pc_hbm,
               src_all, dst_all, rows0, rows1, ea0, ea1, out0, out1,
               t_v, acc,
               gsem0, gsem1, ssem0, ssem1, isem):
    cid = lax.axis_index("c")
    sid = lax.axis_index("s")
    wid = cid * NS + sid
    ebase = wid * EPT
    rbase = wid * NCHK

    rows = (rows0, rows1)
    eav = (ea0, ea1)
    outv = (out0, out1)
    gsem = (gsem0, gsem1)
    ssem = (ssem0, ssem1)

    pltpu.async_copy(src_hbm.at[pl.ds(rbase, NCHK)], src_all, isem)
    pltpu.async_copy(dst_hbm.at[pl.ds(rbase, NCHK)], dst_all, isem)
    pltpu.sync_copy(t_hbm, t_v)

    pltpu.async_copy(z_hbm, acc.at[pl.ds(sid * ZR, ZR)], isem)
    pltpu.make_async_copy(z_hbm, acc.at[pl.ds(sid * ZR, ZR)], isem).wait()
    pltpu.make_async_copy(src_hbm.at[pl.ds(rbase, NCHK)], src_all, isem).wait()
    pltpu.make_async_copy(dst_hbm.at[pl.ds(rbase, NCHK)], dst_all, isem).wait()
    plsc.subcore_barrier()

    t = t_v[...]

    def _issue(c, p):
        pltpu.async_copy(ea_hbm.at[pl.ds(ebase + c * G, G)], eav[p], gsem[p])
        pltpu.async_copy(h_hbm.at[src_all.at[c]], rows[p], gsem[p])

    def _wait_inputs(c, p):
        pltpu.make_async_copy(ea_hbm.at[pl.ds(ebase + c * G, G)], eav[p], gsem[p]).wait()
        pltpu.make_async_copy(h_hbm.at[src_all.at[c]], rows[p], gsem[p]).wait()

    def _issue_scatter(c, p):
        pltpu.async_copy(outv[p], acc.at[dst_all.at[c]], ssem[p], add=True)

    def _wait_scatter(c, p):
        pltpu.make_async_copy(outv[p], acc.at[dst_all.at[c]], ssem[p]).wait()

    _issue(0, 0)
    _issue(1, 1)

    def _process(c, p):
        _wait_inputs(c, p)

        @pl.when(c >= 2)
        def _drain(c=c, p=p):
            _wait_scatter(c - 2, p)

        rv, ev, ov = rows[p], eav[p], outv[p]

        @plsc.parallel_loop(0, G, 1, unroll=8)
        def _edge(e):
            for j in range(4):
                sl = pl.ds(j * 16, 16)
                m = jnp.maximum(rv[e, sl] + ev[e, sl], 0.0) + 1e-7
                s = jnp.exp(m * t)
                ov[e, sl] = s * m
                ov[e, pl.ds(D + j * 16, 16)] = s

        _issue_scatter(c, p)

        @pl.when(c + 2 < NCHK)
        def _prefetch(c=c, p=p):
            _issue(c + 2, p)

    def _step(i, carry):
        _process(2 * i, 0)
        _process(2 * i + 1, 1)
        return carry
    lax.fori_loop(0, NCHK // 2, _step, 0)
    _wait_scatter(NCHK - 2, 0)
    _wait_scatter(NCHK - 1, 1)
    plsc.subcore_barrier()

    pltpu.async_copy(acc.at[pl.ds(sid * ZR, ZR)],
                     pc_hbm.at[cid, pl.ds(sid * ZR, ZR)], isem)
    pltpu.make_async_copy(acc.at[pl.ds(sid * ZR, ZR)],
                          pc_hbm.at[cid, pl.ds(sid * ZR, ZR)], isem).wait()


_edge_call = functools.partial(
    pl.kernel,
    out_type=jax.ShapeDtypeStruct((NC, N, D2), jnp.float32),
    mesh=_mesh,
    compiler_params=pltpu.CompilerParams(use_tc_tiling_on_sc=False),
    scratch_types=[
        pltpu.VMEM((NCHK, G), jnp.int32),
        pltpu.VMEM((NCHK, G), jnp.int32),
        pltpu.VMEM((G, D), jnp.float32),
        pltpu.VMEM((G, D), jnp.float32),
        pltpu.VMEM((G, D), jnp.float32),
        pltpu.VMEM((G, D), jnp.float32),
        pltpu.VMEM((G, D2), jnp.float32),
        pltpu.VMEM((G, D2), jnp.float32),
        pltpu.VMEM((16,), jnp.float32),
        pltpu.VMEM_SHARED((N, D2), jnp.float32),
        pltpu.SemaphoreType.DMA,
        pltpu.SemaphoreType.DMA,
        pltpu.SemaphoreType.DMA,
        pltpu.SemaphoreType.DMA,
        pltpu.SemaphoreType.DMA,
    ],
)(_edge_body)


def _ln(h, g, b, eps=1e-5):
    mu = jnp.mean(h, axis=-1, keepdims=True)
    var = jnp.var(h, axis=-1, keepdims=True)
    return (h - mu) * lax.rsqrt(var + eps) * g + b


NB = 1000


def _encode_body(x_ref, w_ref, b_ref, o_ref):
    o_ref[...] = jnp.dot(x_ref[...], w_ref[...],
                         preferred_element_type=jnp.float32) + b_ref[...]


def _encode(x, w, b):
    rows = x.shape[0]
    return pl.pallas_call(
        _encode_body,
        grid=(rows // NB,),
        in_specs=[
            pl.BlockSpec((NB, D2), lambda i: (i, 0)),
            pl.BlockSpec((D2, D), lambda i: (0, 0)),
            pl.BlockSpec((1, D), lambda i: (0, 0)),
        ],
        out_specs=pl.BlockSpec((NB, D), lambda i: (i, 0)),
        out_shape=jax.ShapeDtypeStruct((rows, D), jnp.float32),
    )(x, w, b)


def _node_body(pc_ref, cin_ref, hprev_ref,
               w1_ref, b1_ref, g1_ref, bt1_ref, w2_ref, b2_ref,
               gn_ref, bn_ref, hout_ref, nin_ref):
    numer = pc_ref[0, :, :D] + pc_ref[1, :, :D]
    denom = pc_ref[0, :, D:] + pc_ref[1, :, D:]
    out = numer / (denom + 1e-16) + cin_ref[...]
    hm = jnp.dot(out, w1_ref[...], preferred_element_type=jnp.float32) + b1_ref[...]
    hm = jax.nn.relu(_ln(hm, g1_ref[...], bt1_ref[...]))
    r = jnp.dot(hm, w2_ref[...], preferred_element_type=jnp.float32) + b2_ref[...]
    h = hprev_ref[...] + r
    hout_ref[...] = h
    nin_ref[...] = jax.nn.relu(_ln(h, gn_ref[...], bn_ref[...]))


def _node(pc, cin, hprev, w1, b1, g1, bt1, w2, b2, gn, bn):
    vec = lambda: pl.BlockSpec((1, D2), lambda i: (0, 0))
    vec64 = lambda: pl.BlockSpec((1, D), lambda i: (0, 0))
    return pl.pallas_call(
        _node_body,
        grid=(N // NB,),
        in_specs=[
            pl.BlockSpec((NC, NB, D2), lambda i: (0, i, 0)),
            pl.BlockSpec((NB, D), lambda i: (i, 0)),
            pl.BlockSpec((NB, D), lambda i: (i, 0)),
            pl.BlockSpec((D, D2), lambda i: (0, 0)),
            vec(), vec(), vec(),
            pl.BlockSpec((D2, D), lambda i: (0, 0)),
            vec64(), vec64(), vec64(),
        ],
        out_specs=[pl.BlockSpec((NB, D), lambda i: (i, 0)),
                   pl.BlockSpec((NB, D), lambda i: (i, 0))],
        out_shape=[jax.ShapeDtypeStruct((N, D), jnp.float32),
                   jax.ShapeDtypeStruct((N, D), jnp.float32)],
    )(pc, cin, hprev, w1, b1, g1, bt1, w2, b2, gn, bn)


def kernel(x, edge_index, edge_attr, W_ne, b_ne, W_ee, b_ee, t, W1, b1, g1, bt1, W2, b2, ln_g, ln_b):
    src = edge_index[0].reshape(E // G, G)
    dst = edge_index[1].reshape(E // G, G)
    h0 = _encode(x, W_ne, b_ne.reshape(1, D))
    ea = _encode(edge_attr, W_ee, b_ee.reshape(1, D))
    tvecs = jnp.broadcast_to(t[:, None], (L, 16)).astype(jnp.float32)
    zblock = jnp.zeros((ZR, D2), jnp.float32)

    cin = h0
    hprev = jnp.zeros((N, D), jnp.float32)
    for i in range(L):
        pc = _edge_call(cin, ea, src, dst, tvecs[i], zblock)
        j = (i + 1) % L
        hprev, cin = _node(pc, cin, hprev,
                           W1[i], b1[i].reshape(1, D2), g1[i].reshape(1, D2),
                           bt1[i].reshape(1, D2), W2[i], b2[i].reshape(1, D),
                           ln_g[j].reshape(1, D), ln_b[j].reshape(1, D))
    return cin


# speedup vs baseline: 1.2124x; 1.0034x over previous
import functools

import jax
import jax.numpy as jnp
from jax import lax
from jax.experimental import pallas as pl
from jax.experimental.pallas import tpu as pltpu
from jax.experimental.pallas import tpu_sc as plsc

N = 10000
E = 320000
D = 64
D2 = 128
L = 14

NC = 2
NS = 16
NW = NC * NS
EPT = E // NW
G = 50
NCHK = EPT // G
ZR = N // NS

_mesh = plsc.VectorSubcoreMesh(core_axis_name="c", subcore_axis_name="s")


def _edge_body(h_hbm, ea_hbm, src_hbm, dst_hbm, t_hbm, z_hbm, pc_hbm,
               src_all, dst_all, rows0, rows1, ea0, ea1, out0, out1,
               t_v, acc,
               gsem0, gsem1, ssem0, ssem1, isem, zsem):
    cid = lax.axis_index("c")
    sid = lax.axis_index("s")
    wid = cid * NS + sid
    ebase = wid * EPT
    rbase = wid * NCHK

    rows = (rows0, rows1)
    eav = (ea0, ea1)
    outv = (out0, out1)
    gsem = (gsem0, gsem1)
    ssem = (ssem0, ssem1)

    pltpu.async_copy(src_hbm.at[pl.ds(rbase, NCHK)], src_all, isem)
    pltpu.async_copy(dst_hbm.at[pl.ds(rbase, NCHK)], dst_all, isem)
    pltpu.sync_copy(t_hbm, t_v)

    pltpu.async_copy(z_hbm, acc.at[pl.ds(sid * ZR, ZR)], zsem)
    pltpu.make_async_copy(src_hbm.at[pl.ds(rbase, NCHK)], src_all, isem).wait()
    pltpu.make_async_copy(dst_hbm.at[pl.ds(rbase, NCHK)], dst_all, isem).wait()

    t = t_v[...]

    def _issue(c, p):
        pltpu.async_copy(ea_hbm.at[pl.ds(ebase + c * G, G)], eav[p], gsem[p])
        pltpu.async_copy(h_hbm.at[src_all.at[c]], rows[p], gsem[p])

    def _wait_inputs(c, p):
        pltpu.make_async_copy(ea_hbm.at[pl.ds(ebase + c * G, G)], eav[p], gsem[p]).wait()
        pltpu.make_async_copy(h_hbm.at[src_all.at[c]], rows[p], gsem[p]).wait()

    def _issue_scatter(c, p):
        pltpu.async_copy(outv[p], acc.at[dst_all.at[c]], ssem[p], add=True)

    def _wait_scatter(c, p):
        pltpu.make_async_copy(outv[p], acc.at[dst_all.at[c]], ssem[p]).wait()

    _issue(0, 0)
    _issue(1, 1)
    pltpu.make_async_copy(z_hbm, acc.at[pl.ds(sid * ZR, ZR)], zsem).wait()
    plsc.subcore_barrier()

    def _process(c, p):
        _wait_inputs(c, p)

        @pl.when(c >= 2)
        def _drain(c=c, p=p):
            _wait_scatter(c - 2, p)

        rv, ev, ov = rows[p], eav[p], outv[p]

        @plsc.parallel_loop(0, G, 1, unroll=8)
        def _edge(e):
            for j in range(4):
                sl = pl.ds(j * 16, 16)
                m = jnp.maximum(rv[e, sl] + ev[e, sl], 0.0) + 1e-7
                s = jnp.exp(m * t)
                ov[e, sl] = s * m
                ov[e, pl.ds(D + j * 16, 16)] = s

        _issue_scatter(c, p)

        @pl.when(c + 2 < NCHK)
        def _prefetch(c=c, p=p):
            _issue(c + 2, p)

    def _step(i, carry):
        _process(2 * i, 0)
        _process(2 * i + 1, 1)
        return carry
    lax.fori_loop(0, NCHK // 2, _step, 0)
    _wait_scatter(NCHK - 2, 0)
    _wait_scatter(NCHK - 1, 1)
    plsc.subcore_barrier()

    pltpu.async_copy(acc.at[pl.ds(sid * ZR, ZR)],
                     pc_hbm.at[cid, pl.ds(sid * ZR, ZR)], isem)
    pltpu.make_async_copy(acc.at[pl.ds(sid * ZR, ZR)],
                          pc_hbm.at[cid, pl.ds(sid * ZR, ZR)], isem).wait()


_edge_call = functools.partial(
    pl.kernel,
    out_type=jax.ShapeDtypeStruct((NC, N, D2), jnp.float32),
    mesh=_mesh,
    compiler_params=pltpu.CompilerParams(use_tc_tiling_on_sc=False),
    scratch_types=[
        pltpu.VMEM((NCHK, G), jnp.int32),
        pltpu.VMEM((NCHK, G), jnp.int32),
        pltpu.VMEM((G, D), jnp.float32),
        pltpu.VMEM((G, D), jnp.float32),
        pltpu.VMEM((G, D), jnp.float32),
        pltpu.VMEM((G, D), jnp.float32),
        pltpu.VMEM((G, D2), jnp.float32),
        pltpu.VMEM((G, D2), jnp.float32),
        pltpu.VMEM((16,), jnp.float32),
        pltpu.VMEM_SHARED((N, D2), jnp.float32),
        pltpu.SemaphoreType.DMA,
        pltpu.SemaphoreType.DMA,
        pltpu.SemaphoreType.DMA,
        pltpu.SemaphoreType.DMA,
        pltpu.SemaphoreType.DMA,
        pltpu.SemaphoreType.DMA,
    ],
)(_edge_body)


def _ln(h, g, b, eps=1e-5):
    mu = jnp.mean(h, axis=-1, keepdims=True)
    var = jnp.var(h, axis=-1, keepdims=True)
    return (h - mu) * lax.rsqrt(var + eps) * g + b


NB = 1000


def _encode_body(x_ref, w_ref, b_ref, o_ref):
    o_ref[...] = jnp.dot(x_ref[...], w_ref[...],
                         preferred_element_type=jnp.float32) + b_ref[...]


def _encode(x, w, b):
    rows = x.shape[0]
    return pl.pallas_call(
        _encode_body,
        grid=(rows // NB,),
        in_specs=[
            pl.BlockSpec((NB, D2), lambda i: (i, 0)),
            pl.BlockSpec((D2, D), lambda i: (0, 0)),
            pl.BlockSpec((1, D), lambda i: (0, 0)),
        ],
        out_specs=pl.BlockSpec((NB, D), lambda i: (i, 0)),
        out_shape=jax.ShapeDtypeStruct((rows, D), jnp.float32),
    )(x, w, b)


def _node_body(pc_ref, cin_ref, hprev_ref,
               w1_ref, b1_ref, g1_ref, bt1_ref, w2_ref, b2_ref,
               gn_ref, bn_ref, hout_ref, nin_ref):
    numer = pc_ref[0, :, :D] + pc_ref[1, :, :D]
    denom = pc_ref[0, :, D:] + pc_ref[1, :, D:]
    out = numer / (denom + 1e-16) + cin_ref[...]
    hm = jnp.dot(out, w1_ref[...], preferred_element_type=jnp.float32) + b1_ref[...]
    hm = jax.nn.relu(_ln(hm, g1_ref[...], bt1_ref[...]))
    r = jnp.dot(hm, w2_ref[...], preferred_element_type=jnp.float32) + b2_ref[...]
    h = hprev_ref[...] + r
    hout_ref[...] = h
    nin_ref[...] = jax.nn.relu(_ln(h, gn_ref[...], bn_ref[...]))


def _node(pc, cin, hprev, w1, b1, g1, bt1, w2, b2, gn, bn):
    vec = lambda: pl.BlockSpec((1, D2), lambda i: (0, 0))
    vec64 = lambda: pl.BlockSpec((1, D), lambda i: (0, 0))
    return pl.pallas_call(
        _node_body,
        grid=(N // NB,),
        in_specs=[
            pl.BlockSpec((NC, NB, D2), lambda i: (0, i, 0)),
            pl.BlockSpec((NB, D), lambda i: (i, 0)),
            pl.BlockSpec((NB, D), lambda i: (i, 0)),
            pl.BlockSpec((D, D2), lambda i: (0, 0)),
            vec(), vec(), vec(),
            pl.BlockSpec((D2, D), lambda i: (0, 0)),
            vec64(), vec64(), vec64(),
        ],
        out_specs=[pl.BlockSpec((NB, D), lambda i: (i, 0)),
                   pl.BlockSpec((NB, D), lambda i: (i, 0))],
        out_shape=[jax.ShapeDtypeStruct((N, D), jnp.float32),
                   jax.ShapeDtypeStruct((N, D), jnp.float32)],
    )(pc, cin, hprev, w1, b1, g1, bt1, w2, b2, gn, bn)


def kernel(x, edge_index, edge_attr, W_ne, b_ne, W_ee, b_ee, t, W1, b1, g1, bt1, W2, b2, ln_g, ln_b):
    src = edge_index[0].reshape(E // G, G)
    dst = edge_index[1].reshape(E // G, G)
    h0 = _encode(x, W_ne, b_ne.reshape(1, D))
    ea = _encode(edge_attr, W_ee, b_ee.reshape(1, D))
    tvecs = jnp.broadcast_to(t[:, None], (L, 16)).astype(jnp.float32)
    zblock = jnp.zeros((ZR, D2), jnp.float32)

    cin = h0
    hprev = jnp.zeros((N, D), jnp.float32)
    for i in range(L):
        pc = _edge_call(cin, ea, src, dst, tvecs[i], zblock)
        j = (i + 1) % L
        hprev, cin = _node(pc, cin, hprev,
                           W1[i], b1[i].reshape(1, D2), g1[i].reshape(1, D2),
                           bt1[i].reshape(1, D2), W2[i], b2[i].reshape(1, D),
                           ln_g[j].reshape(1, D), ln_b[j].reshape(1, D))
    return cin
